# baseline (speedup 1.0000x reference)
import functools

import jax
import jax.numpy as jnp
from jax import lax
from jax.experimental import pallas as pl
from jax.experimental.pallas import tpu as pltpu
from jax.experimental.pallas import tpu_sc as plsc

N = 10000
E = 320000
D = 128
NC = 2
NS = 16
K = 50
EPT = E // (NC * NS)
NCH = EPT // K
NHALF = 5
HNCH = NCH // NHALF
SLAB = 640
NFULL = N // SLAB
REM = N - NFULL * SLAB
NBUF = 4

_mesh = plsc.VectorSubcoreMesh(core_axis_name="c", subcore_axis_name="s")


@functools.partial(
    pl.kernel,
    out_type=jax.ShapeDtypeStruct((NC, N, D), jnp.float32),
    mesh=_mesh,
    scratch_types=[
        pltpu.VMEM((HNCH, K), jnp.int32),
        pltpu.VMEM((HNCH, K), jnp.int32),
        pltpu.VMEM((NBUF, K, D), jnp.float32),
        pltpu.VMEM_SHARED((N, D), jnp.float32),
        pltpu.SemaphoreType.DMA((NBUF,)),
        pltpu.SemaphoreType.DMA((NBUF,)),
    ],
)
def _sc_aggregate(g_hbm, src_hbm, dst_hbm, zeros_hbm, out_hbm,
                  src_v, dst_v, rows_v, acc_sh, gsem, ssem):
    c = lax.axis_index("c")
    s = lax.axis_index("s")
    base = pl.multiple_of(s * SLAB, 8)

    @pl.when(s < NFULL)
    def _():
        pltpu.sync_copy(zeros_hbm.at[pl.ds(base, SLAB)], acc_sh.at[pl.ds(base, SLAB)])

    @pl.when(s == NFULL)
    def _():
        pltpu.sync_copy(
            zeros_hbm.at[pl.ds(NFULL * SLAB, REM)],
            acc_sh.at[pl.ds(NFULL * SLAB, REM)],
        )

    plsc.subcore_barrier()

    def gather(cj, b):
        pltpu.async_copy(g_hbm.at[src_v.at[cj]], rows_v.at[b], gsem.at[b])

    def gather_wait(cj, b):
        pltpu.make_async_copy(g_hbm.at[src_v.at[cj]], rows_v.at[b], gsem.at[b]).wait()

    def scatter_add_wait(cj, b):
        pltpu.async_copy(
            rows_v.at[b], acc_sh.at[dst_v.at[cj]], ssem.at[b], add=True
        ).wait()

    for h in range(NHALF):
        pltpu.sync_copy(src_hbm.at[c, s, pl.ds(h * HNCH, HNCH)], src_v)
        pltpu.sync_copy(dst_hbm.at[c, s, pl.ds(h * HNCH, HNCH)], dst_v)

        for b in range(NBUF):
            gather(b, b)

        @pl.loop(0, HNCH - NBUF, step=NBUF)
        def _(j):
            for b in range(NBUF):
                cj = j + b
                gather_wait(cj, b)
                scatter_add_wait(cj, b)
                gather(cj + NBUF, b)

        for b in range(NBUF):
            cj = HNCH - NBUF + b
            gather_wait(cj, b)
            scatter_add_wait(cj, b)

    plsc.subcore_barrier()

    @pl.when(s < NFULL)
    def _():
        pltpu.sync_copy(acc_sh.at[pl.ds(base, SLAB)], out_hbm.at[c, pl.ds(base, SLAB)])

    @pl.when(s == NFULL)
    def _():
        pltpu.sync_copy(
            acc_sh.at[pl.ds(NFULL * SLAB, REM)],
            out_hbm.at[c, pl.ds(NFULL * SLAB, REM)],
        )


_R = 2000


def _mm_body(x_ref, w_ref, h_ref):
    h_ref[...] = jnp.dot(x_ref[...], w_ref[...], preferred_element_type=jnp.float32)


_mm_call = pl.pallas_call(
    _mm_body,
    grid=(N // _R,),
    in_specs=[
        pl.BlockSpec((_R, D), lambda i: (i, 0)),
        pl.BlockSpec((D, D), lambda i: (0, 0)),
    ],
    out_specs=pl.BlockSpec((_R, D), lambda i: (i, 0)),
    out_shape=jax.ShapeDtypeStruct((N, D), jnp.float32),
)


def _pre_body(deg_ref, h_ref, dinv_ref, g_ref):
    deg = deg_ref[0][:, :1] + deg_ref[1][:, :1]
    dinv = lax.rsqrt(deg + 1.0)
    dinv_ref[...] = dinv
    g_ref[...] = h_ref[...] * dinv


_pre_call = pl.pallas_call(
    _pre_body,
    grid=(N // _R,),
    in_specs=[
        pl.BlockSpec((NC, _R, D), lambda i: (0, i, 0)),
        pl.BlockSpec((_R, D), lambda i: (i, 0)),
    ],
    out_specs=[
        pl.BlockSpec((_R, 1), lambda i: (i, 0)),
        pl.BlockSpec((_R, D), lambda i: (i, 0)),
    ],
    out_shape=[
        jax.ShapeDtypeStruct((N, 1), jnp.float32),
        jax.ShapeDtypeStruct((N, D), jnp.float32),
    ],
)


def _mid_body(parts_ref, g_ref, dinv_ref, b_ref, w_ref, gn_ref):
    p = parts_ref[0] + parts_ref[1] + g_ref[...]
    y = jnp.maximum(dinv_ref[...] * p + b_ref[...], 0.0)
    gn = jnp.dot(y, w_ref[...], preferred_element_type=jnp.float32)
    gn_ref[...] = gn * dinv_ref[...]


_mid_call = pl.pallas_call(
    _mid_body,
    grid=(N // _R,),
    in_specs=[
        pl.BlockSpec((NC, _R, D), lambda i: (0, i, 0)),
        pl.BlockSpec((_R, D), lambda i: (i, 0)),
        pl.BlockSpec((_R, 1), lambda i: (i, 0)),
        pl.BlockSpec((1, D), lambda i: (0, 0)),
        pl.BlockSpec((D, D), lambda i: (0, 0)),
    ],
    out_specs=pl.BlockSpec((_R, D), lambda i: (i, 0)),
    out_shape=jax.ShapeDtypeStruct((N, D), jnp.float32),
)


def _post_body(parts_ref, g_ref, dinv_ref, b_ref, out_ref):
    p = parts_ref[0] + parts_ref[1] + g_ref[...]
    out_ref[...] = dinv_ref[...] * p + b_ref[...]


_post_call = pl.pallas_call(
    _post_body,
    grid=(N // _R,),
    in_specs=[
        pl.BlockSpec((NC, _R, D), lambda i: (0, i, 0)),
        pl.BlockSpec((_R, D), lambda i: (i, 0)),
        pl.BlockSpec((_R, 1), lambda i: (i, 0)),
        pl.BlockSpec((1, D), lambda i: (0, 0)),
    ],
    out_specs=pl.BlockSpec((_R, D), lambda i: (i, 0)),
    out_shape=jax.ShapeDtypeStruct((N, D), jnp.float32),
)


def kernel(x, edge_index, W1, b1, W2, b2, W3, b3):
    src = edge_index[0].reshape(NC, NS, NCH, K)
    dst = edge_index[1].reshape(NC, NS, NCH, K)
    zeros = jnp.zeros((N, D), jnp.float32)
    ones_nd = jnp.ones((N, D), jnp.float32)

    degparts = _sc_aggregate(ones_nd, src, dst, zeros)
    h1 = _mm_call(x, W1)
    dinv, g1 = _pre_call(degparts, h1)
    parts1 = _sc_aggregate(g1, src, dst, zeros)
    g2 = _mid_call(parts1, g1, dinv, b1.reshape(1, D), W2)
    parts2 = _sc_aggregate(g2, src, dst, zeros)
    g3 = _mid_call(parts2, g2, dinv, b2.reshape(1, D), W3)
    parts3 = _sc_aggregate(g3, src, dst, zeros)
    return _post_call(parts3, g3, dinv, b3.reshape(1, D))


# speedup vs baseline: 1.0798x; 1.0798x over previous
import functools

import jax
import jax.numpy as jnp
from jax import lax
from jax.experimental import pallas as pl
from jax.experimental.pallas import tpu as pltpu
from jax.experimental.pallas import tpu_sc as plsc

N = 10000
E = 320000
D = 128
NC = 2
NS = 16
K = 50
EPT = E // (NC * NS)
NCH = EPT // K
NHALF = 5
HNCH = NCH // NHALF
SLAB = 640
NFULL = N // SLAB
REM = N - NFULL * SLAB
NBUF = 4

_mesh = plsc.VectorSubcoreMesh(core_axis_name="c", subcore_axis_name="s")


@functools.partial(
    pl.kernel,
    out_type=jax.ShapeDtypeStruct((NC, N, D), jnp.float32),
    mesh=_mesh,
    scratch_types=[
        pltpu.VMEM((HNCH, K), jnp.int32),
        pltpu.VMEM((HNCH, K), jnp.int32),
        pltpu.VMEM((NBUF, K, D), jnp.float32),
        pltpu.VMEM_SHARED((N, D), jnp.float32),
        pltpu.SemaphoreType.DMA((NBUF,)),
        pltpu.SemaphoreType.DMA((NBUF,)),
    ],
)
def _sc_aggregate(g_hbm, src_hbm, dst_hbm, zeros_hbm, out_hbm,
                  src_v, dst_v, rows_v, acc_sh, gsem, ssem):
    c = lax.axis_index("c")
    s = lax.axis_index("s")
    base = pl.multiple_of(s * SLAB, 8)

    @pl.when(s < NFULL)
    def _():
        pltpu.sync_copy(zeros_hbm.at[pl.ds(base, SLAB)], acc_sh.at[pl.ds(base, SLAB)])

    @pl.when(s == NFULL)
    def _():
        pltpu.sync_copy(
            zeros_hbm.at[pl.ds(NFULL * SLAB, REM)],
            acc_sh.at[pl.ds(NFULL * SLAB, REM)],
        )

    plsc.subcore_barrier()

    def gather(cj, b):
        pltpu.async_copy(g_hbm.at[src_v.at[cj]], rows_v.at[b], gsem.at[b])

    def gather_wait(cj, b):
        pltpu.make_async_copy(g_hbm.at[src_v.at[cj]], rows_v.at[b], gsem.at[b]).wait()

    def scatter_add_wait(cj, b):
        pltpu.async_copy(
            rows_v.at[b], acc_sh.at[dst_v.at[cj]], ssem.at[b], add=True
        ).wait()

    for h in range(NHALF):
        pltpu.sync_copy(src_hbm.at[c, s, pl.ds(h * HNCH, HNCH)], src_v)
        pltpu.sync_copy(dst_hbm.at[c, s, pl.ds(h * HNCH, HNCH)], dst_v)

        for b in range(NBUF):
            gather(b, b)

        @pl.loop(0, HNCH - NBUF, step=NBUF)
        def _(j):
            for b in range(NBUF):
                cj = j + b
                gather_wait(cj, b)
                scatter_add_wait(cj, b)
                gather(cj + NBUF, b)

        for b in range(NBUF):
            cj = HNCH - NBUF + b
            gather_wait(cj, b)
            scatter_add_wait(cj, b)

    plsc.subcore_barrier()

    @pl.when(s < NFULL)
    def _():
        pltpu.sync_copy(acc_sh.at[pl.ds(base, SLAB)], out_hbm.at[c, pl.ds(base, SLAB)])

    @pl.when(s == NFULL)
    def _():
        pltpu.sync_copy(
            acc_sh.at[pl.ds(NFULL * SLAB, REM)],
            out_hbm.at[c, pl.ds(NFULL * SLAB, REM)],
        )


@functools.partial(
    pl.kernel,
    out_type=jax.ShapeDtypeStruct((NC, N, D), jnp.float32),
    mesh=_mesh,
    scratch_types=[
        pltpu.VMEM((HNCH, K), jnp.int32),
        pltpu.VMEM((K, D), jnp.float32),
        pltpu.VMEM_SHARED((N, D), jnp.float32),
        pltpu.SemaphoreType.DMA,
        pltpu.SemaphoreType.DMA((NBUF,)),
    ],
)
def _sc_degree(ones_hbm, dst_hbm, zeros_hbm, out_hbm,
               dst_v, rows_v, acc_sh, gsem, ssem):
    c = lax.axis_index("c")
    s = lax.axis_index("s")
    base = pl.multiple_of(s * SLAB, 8)

    @pl.when(s < NFULL)
    def _():
        pltpu.sync_copy(zeros_hbm.at[pl.ds(base, SLAB)], acc_sh.at[pl.ds(base, SLAB)])

    @pl.when(s == NFULL)
    def _():
        pltpu.sync_copy(
            zeros_hbm.at[pl.ds(NFULL * SLAB, REM)],
            acc_sh.at[pl.ds(NFULL * SLAB, REM)],
        )

    pltpu.sync_copy(dst_hbm.at[c, s, pl.ds(0, HNCH)], dst_v)
    pltpu.async_copy(ones_hbm.at[dst_v.at[0]], rows_v, gsem).wait()
    plsc.subcore_barrier()

    def scatter_add(cj, b):
        pltpu.async_copy(rows_v, acc_sh.at[dst_v.at[cj]], ssem.at[b], add=True)

    def scatter_wait(cj, b):
        pltpu.make_async_copy(rows_v, acc_sh.at[dst_v.at[cj]], ssem.at[b]).wait()

    for h in range(NHALF):
        if h > 0:
            pltpu.sync_copy(dst_hbm.at[c, s, pl.ds(h * HNCH, HNCH)], dst_v)

        for b in range(NBUF):
            scatter_add(b, b)

        @pl.loop(0, HNCH - NBUF, step=NBUF)
        def _(j):
            for b in range(NBUF):
                cj = j + b
                scatter_wait(cj, b)
                scatter_add(cj + NBUF, b)

        for b in range(NBUF):
            scatter_wait(HNCH - NBUF + b, b)

    plsc.subcore_barrier()

    @pl.when(s < NFULL)
    def _():
        pltpu.sync_copy(acc_sh.at[pl.ds(base, SLAB)], out_hbm.at[c, pl.ds(base, SLAB)])

    @pl.when(s == NFULL)
    def _():
        pltpu.sync_copy(
            acc_sh.at[pl.ds(NFULL * SLAB, REM)],
            out_hbm.at[c, pl.ds(NFULL * SLAB, REM)],
        )


_R = 2000


def _pre_body(deg_ref, x_ref, w_ref, dinv_ref, g_ref):
    deg = deg_ref[0][:, :1] + deg_ref[1][:, :1]
    dinv = lax.rsqrt(deg + 1.0)
    h = jnp.dot(x_ref[...], w_ref[...], preferred_element_type=jnp.float32)
    dinv_ref[...] = dinv
    g_ref[...] = h * dinv


_pre_call = pl.pallas_call(
    _pre_body,
    grid=(N // _R,),
    in_specs=[
        pl.BlockSpec((NC, _R, D), lambda i: (0, i, 0)),
        pl.BlockSpec((_R, D), lambda i: (i, 0)),
        pl.BlockSpec((D, D), lambda i: (0, 0)),
    ],
    out_specs=[
        pl.BlockSpec((_R, 1), lambda i: (i, 0)),
        pl.BlockSpec((_R, D), lambda i: (i, 0)),
    ],
    out_shape=[
        jax.ShapeDtypeStruct((N, 1), jnp.float32),
        jax.ShapeDtypeStruct((N, D), jnp.float32),
    ],
)


def _mid_body(parts_ref, g_ref, dinv_ref, b_ref, w_ref, gn_ref):
    p = parts_ref[0] + parts_ref[1] + g_ref[...]
    y = jnp.maximum(dinv_ref[...] * p + b_ref[...], 0.0)
    gn = jnp.dot(y, w_ref[...], preferred_element_type=jnp.float32)
    gn_ref[...] = gn * dinv_ref[...]


_mid_call = pl.pallas_call(
    _mid_body,
    grid=(N // _R,),
    in_specs=[
        pl.BlockSpec((NC, _R, D), lambda i: (0, i, 0)),
        pl.BlockSpec((_R, D), lambda i: (i, 0)),
        pl.BlockSpec((_R, 1), lambda i: (i, 0)),
        pl.BlockSpec((1, D), lambda i: (0, 0)),
        pl.BlockSpec((D, D), lambda i: (0, 0)),
    ],
    out_specs=pl.BlockSpec((_R, D), lambda i: (i, 0)),
    out_shape=jax.ShapeDtypeStruct((N, D), jnp.float32),
)


def _post_body(parts_ref, g_ref, dinv_ref, b_ref, out_ref):
    p = parts_ref[0] + parts_ref[1] + g_ref[...]
    out_ref[...] = dinv_ref[...] * p + b_ref[...]


_post_call = pl.pallas_call(
    _post_body,
    grid=(N // _R,),
    in_specs=[
        pl.BlockSpec((NC, _R, D), lambda i: (0, i, 0)),
        pl.BlockSpec((_R, D), lambda i: (i, 0)),
        pl.BlockSpec((_R, 1), lambda i: (i, 0)),
        pl.BlockSpec((1, D), lambda i: (0, 0)),
    ],
    out_specs=pl.BlockSpec((_R, D), lambda i: (i, 0)),
    out_shape=jax.ShapeDtypeStruct((N, D), jnp.float32),
)


def kernel(x, edge_index, W1, b1, W2, b2, W3, b3):
    src = edge_index[0].reshape(NC, NS, NCH, K)
    dst = edge_index[1].reshape(NC, NS, NCH, K)
    zeros = jnp.zeros((N, D), jnp.float32)
    ones_nd = jnp.ones((N, D), jnp.float32)

    degparts = _sc_degree(ones_nd, dst, zeros)
    dinv, g1 = _pre_call(degparts, x, W1)
    parts1 = _sc_aggregate(g1, src, dst, zeros)
    g2 = _mid_call(parts1, g1, dinv, b1.reshape(1, D), W2)
    parts2 = _sc_aggregate(g2, src, dst, zeros)
    g3 = _mid_call(parts2, g2, dinv, b2.reshape(1, D), W3)
    parts3 = _sc_aggregate(g3, src, dst, zeros)
    return _post_call(parts3, g3, dinv, b3.reshape(1, D))


# speedup vs baseline: 1.0810x; 1.0011x over previous
import functools

import jax
import jax.numpy as jnp
from jax import lax
from jax.experimental import pallas as pl
from jax.experimental.pallas import tpu as pltpu
from jax.experimental.pallas import tpu_sc as plsc

N = 10000
E = 320000
D = 128
NC = 2
NS = 16
K = 50
EPT = E // (NC * NS)
NCH = EPT // K
NHALF = 5
HNCH = NCH // NHALF
SLAB = 640
NFULL = N // SLAB
REM = N - NFULL * SLAB
NBUF = 5

_mesh = plsc.VectorSubcoreMesh(core_axis_name="c", subcore_axis_name="s")


@functools.partial(
    pl.kernel,
    out_type=jax.ShapeDtypeStruct((NC, N, D), jnp.float32),
    mesh=_mesh,
    scratch_types=[
        pltpu.VMEM((HNCH, K), jnp.int32),
        pltpu.VMEM((HNCH, K), jnp.int32),
        pltpu.VMEM((NBUF, K, D), jnp.float32),
        pltpu.VMEM_SHARED((N, D), jnp.float32),
        pltpu.SemaphoreType.DMA((NBUF,)),
        pltpu.SemaphoreType.DMA((NBUF,)),
    ],
)
def _sc_aggregate(g_hbm, src_hbm, dst_hbm, zeros_hbm, out_hbm,
                  src_v, dst_v, rows_v, acc_sh, gsem, ssem):
    c = lax.axis_index("c")
    s = lax.axis_index("s")
    base = pl.multiple_of(s * SLAB, 8)

    @pl.when(s < NFULL)
    def _():
        pltpu.sync_copy(zeros_hbm.at[pl.ds(base, SLAB)], acc_sh.at[pl.ds(base, SLAB)])

    @pl.when(s == NFULL)
    def _():
        pltpu.sync_copy(
            zeros_hbm.at[pl.ds(NFULL * SLAB, REM)],
            acc_sh.at[pl.ds(NFULL * SLAB, REM)],
        )

    plsc.subcore_barrier()

    def gather(cj, b):
        pltpu.async_copy(g_hbm.at[src_v.at[cj]], rows_v.at[b], gsem.at[b])

    def gather_wait(cj, b):
        pltpu.make_async_copy(g_hbm.at[src_v.at[cj]], rows_v.at[b], gsem.at[b]).wait()

    def scatter_add_wait(cj, b):
        pltpu.async_copy(
            rows_v.at[b], acc_sh.at[dst_v.at[cj]], ssem.at[b], add=True
        ).wait()

    for h in range(NHALF):
        pltpu.sync_copy(src_hbm.at[c, s, pl.ds(h * HNCH, HNCH)], src_v)
        pltpu.sync_copy(dst_hbm.at[c, s, pl.ds(h * HNCH, HNCH)], dst_v)

        for b in range(NBUF):
            gather(b, b)

        @pl.loop(0, HNCH - NBUF, step=NBUF)
        def _(j):
            for b in range(NBUF):
                cj = j + b
                gather_wait(cj, b)
                scatter_add_wait(cj, b)
                gather(cj + NBUF, b)

        for b in range(NBUF):
            cj = HNCH - NBUF + b
            gather_wait(cj, b)
            scatter_add_wait(cj, b)

    plsc.subcore_barrier()

    @pl.when(s < NFULL)
    def _():
        pltpu.sync_copy(acc_sh.at[pl.ds(base, SLAB)], out_hbm.at[c, pl.ds(base, SLAB)])

    @pl.when(s == NFULL)
    def _():
        pltpu.sync_copy(
            acc_sh.at[pl.ds(NFULL * SLAB, REM)],
            out_hbm.at[c, pl.ds(NFULL * SLAB, REM)],
        )


@functools.partial(
    pl.kernel,
    out_type=jax.ShapeDtypeStruct((NC, N, D), jnp.float32),
    mesh=_mesh,
    scratch_types=[
        pltpu.VMEM((HNCH, K), jnp.int32),
        pltpu.VMEM((K, D), jnp.float32),
        pltpu.VMEM_SHARED((N, D), jnp.float32),
        pltpu.SemaphoreType.DMA,
        pltpu.SemaphoreType.DMA((NBUF,)),
    ],
)
def _sc_degree(ones_hbm, dst_hbm, zeros_hbm, out_hbm,
               dst_v, rows_v, acc_sh, gsem, ssem):
    c = lax.axis_index("c")
    s = lax.axis_index("s")
    base = pl.multiple_of(s * SLAB, 8)

    @pl.when(s < NFULL)
    def _():
        pltpu.sync_copy(zeros_hbm.at[pl.ds(base, SLAB)], acc_sh.at[pl.ds(base, SLAB)])

    @pl.when(s == NFULL)
    def _():
        pltpu.sync_copy(
            zeros_hbm.at[pl.ds(NFULL * SLAB, REM)],
            acc_sh.at[pl.ds(NFULL * SLAB, REM)],
        )

    pltpu.sync_copy(dst_hbm.at[c, s, pl.ds(0, HNCH)], dst_v)
    pltpu.async_copy(ones_hbm.at[dst_v.at[0]], rows_v, gsem).wait()
    plsc.subcore_barrier()

    def scatter_add(cj, b):
        pltpu.async_copy(rows_v, acc_sh.at[dst_v.at[cj]], ssem.at[b], add=True)

    def scatter_wait(cj, b):
        pltpu.make_async_copy(rows_v, acc_sh.at[dst_v.at[cj]], ssem.at[b]).wait()

    for h in range(NHALF):
        if h > 0:
            pltpu.sync_copy(dst_hbm.at[c, s, pl.ds(h * HNCH, HNCH)], dst_v)

        for b in range(NBUF):
            scatter_add(b, b)

        @pl.loop(0, HNCH - NBUF, step=NBUF)
        def _(j):
            for b in range(NBUF):
                cj = j + b
                scatter_wait(cj, b)
                scatter_add(cj + NBUF, b)

        for b in range(NBUF):
            scatter_wait(HNCH - NBUF + b, b)

    plsc.subcore_barrier()

    @pl.when(s < NFULL)
    def _():
        pltpu.sync_copy(acc_sh.at[pl.ds(base, SLAB)], out_hbm.at[c, pl.ds(base, SLAB)])

    @pl.when(s == NFULL)
    def _():
        pltpu.sync_copy(
            acc_sh.at[pl.ds(NFULL * SLAB, REM)],
            out_hbm.at[c, pl.ds(NFULL * SLAB, REM)],
        )


_R = 2000


def _pre_body(deg_ref, x_ref, w_ref, dinv_ref, g_ref):
    deg = deg_ref[0][:, :1] + deg_ref[1][:, :1]
    dinv = lax.rsqrt(deg + 1.0)
    h = jnp.dot(x_ref[...], w_ref[...], preferred_element_type=jnp.float32)
    dinv_ref[...] = dinv
    g_ref[...] = h * dinv


_pre_call = pl.pallas_call(
    _pre_body,
    grid=(N // _R,),
    in_specs=[
        pl.BlockSpec((NC, _R, D), lambda i: (0, i, 0)),
        pl.BlockSpec((_R, D), lambda i: (i, 0)),
        pl.BlockSpec((D, D), lambda i: (0, 0)),
    ],
    out_specs=[
        pl.BlockSpec((_R, 1), lambda i: (i, 0)),
        pl.BlockSpec((_R, D), lambda i: (i, 0)),
    ],
    out_shape=[
        jax.ShapeDtypeStruct((N, 1), jnp.float32),
        jax.ShapeDtypeStruct((N, D), jnp.float32),
    ],
)


def _mid_body(parts_ref, g_ref, dinv_ref, b_ref, w_ref, gn_ref):
    p = parts_ref[0] + parts_ref[1] + g_ref[...]
    y = jnp.maximum(dinv_ref[...] * p + b_ref[...], 0.0)
    gn = jnp.dot(y, w_ref[...], preferred_element_type=jnp.float32)
    gn_ref[...] = gn * dinv_ref[...]


_mid_call = pl.pallas_call(
    _mid_body,
    grid=(N // _R,),
    in_specs=[
        pl.BlockSpec((NC, _R, D), lambda i: (0, i, 0)),
        pl.BlockSpec((_R, D), lambda i: (i, 0)),
        pl.BlockSpec((_R, 1), lambda i: (i, 0)),
        pl.BlockSpec((1, D), lambda i: (0, 0)),
        pl.BlockSpec((D, D), lambda i: (0, 0)),
    ],
    out_specs=pl.BlockSpec((_R, D), lambda i: (i, 0)),
    out_shape=jax.ShapeDtypeStruct((N, D), jnp.float32),
)


def _post_body(parts_ref, g_ref, dinv_ref, b_ref, out_ref):
    p = parts_ref[0] + parts_ref[1] + g_ref[...]
    out_ref[...] = dinv_ref[...] * p + b_ref[...]


_post_call = pl.pallas_call(
    _post_body,
    grid=(N // _R,),
    in_specs=[
        pl.BlockSpec((NC, _R, D), lambda i: (0, i, 0)),
        pl.BlockSpec((_R, D), lambda i: (i, 0)),
        pl.BlockSpec((_R, 1), lambda i: (i, 0)),
        pl.BlockSpec((1, D), lambda i: (0, 0)),
    ],
    out_specs=pl.BlockSpec((_R, D), lambda i: (i, 0)),
    out_shape=jax.ShapeDtypeStruct((N, D), jnp.float32),
)


def kernel(x, edge_index, W1, b1, W2, b2, W3, b3):
    src = edge_index[0].reshape(NC, NS, NCH, K)
    dst = edge_index[1].reshape(NC, NS, NCH, K)
    zeros = jnp.zeros((N, D), jnp.float32)
    ones_nd = jnp.ones((N, D), jnp.float32)

    degparts = _sc_degree(ones_nd, dst, zeros)
    dinv, g1 = _pre_call(degparts, x, W1)
    parts1 = _sc_aggregate(g1, src, dst, zeros)
    g2 = _mid_call(parts1, g1, dinv, b1.reshape(1, D), W2)
    parts2 = _sc_aggregate(g2, src, dst, zeros)
    g3 = _mid_call(parts2, g2, dinv, b2.reshape(1, D), W3)
    parts3 = _sc_aggregate(g3, src, dst, zeros)
    return _post_call(parts3, g3, dinv, b3.reshape(1, D))


# speedup vs baseline: 1.1160x; 1.0324x over previous
import functools

import jax
import jax.numpy as jnp
from jax import lax
from jax.experimental import pallas as pl
from jax.experimental.pallas import tpu as pltpu
from jax.experimental.pallas import tpu_sc as plsc

N = 10000
E = 320000
D = 128
NC = 2
NS = 16
K = 50
EPT = E // (NC * NS)
NCH = EPT // K
NHALF = 5
HNCH = NCH // NHALF
SLAB = 640
NFULL = N // SLAB
REM = N - NFULL * SLAB
NBUF = 4
ZROWS = 40

_mesh = plsc.VectorSubcoreMesh(core_axis_name="c", subcore_axis_name="s")


def _zero_init(s, base, zbuf_v, acc_sh, sem):
    zeros16 = jnp.zeros((16,), jnp.float32)

    def fill(i, carry):
        for kk in range(D // 16):
            zbuf_v[i, pl.ds(kk * 16, 16)] = zeros16
        return carry

    lax.fori_loop(0, ZROWS, fill, 0)
    nslab = SLAB // ZROWS
    nrem = REM // ZROWS

    @pl.when(s < NFULL)
    def _():
        for jb in range(0, nslab, 4):
            for j in range(jb, jb + 4):
                off = pl.multiple_of(base + j * ZROWS, 8)
                pltpu.async_copy(zbuf_v, acc_sh.at[pl.ds(off, ZROWS)], sem)
            for j in range(4):
                pltpu.make_async_copy(zbuf_v, acc_sh.at[pl.ds(base, ZROWS)], sem).wait()

    @pl.when(s == NFULL)
    def _():
        for jb in range(0, nrem, 5):
            for j in range(jb, jb + 5):
                off = NFULL * SLAB + j * ZROWS
                pltpu.async_copy(zbuf_v, acc_sh.at[pl.ds(off, ZROWS)], sem)
            for j in range(5):
                pltpu.make_async_copy(
                    zbuf_v, acc_sh.at[pl.ds(NFULL * SLAB, ZROWS)], sem
                ).wait()


@functools.partial(
    pl.kernel,
    out_type=jax.ShapeDtypeStruct((NC, N, D), jnp.float32),
    mesh=_mesh,
    scratch_types=[
        pltpu.VMEM((HNCH, K), jnp.int32),
        pltpu.VMEM((HNCH, K), jnp.int32),
        pltpu.VMEM((NBUF, K, D), jnp.float32),
        pltpu.VMEM((ZROWS, D), jnp.float32),
        pltpu.VMEM_SHARED((N, D), jnp.float32),
        pltpu.SemaphoreType.DMA((NBUF,)),
        pltpu.SemaphoreType.DMA((NBUF,)),
    ],
)
def _sc_aggregate(g_hbm, src_hbm, dst_hbm, out_hbm,
                  src_v, dst_v, rows_v, zbuf_v, acc_sh, gsem, ssem):
    c = lax.axis_index("c")
    s = lax.axis_index("s")
    base = pl.multiple_of(s * SLAB, 8)
    _zero_init(s, base, zbuf_v, acc_sh, gsem.at[0])
    plsc.subcore_barrier()

    def gather(cj, b):
        pltpu.async_copy(g_hbm.at[src_v.at[cj]], rows_v.at[b], gsem.at[b])

    def gather_wait(cj, b):
        pltpu.make_async_copy(g_hbm.at[src_v.at[cj]], rows_v.at[b], gsem.at[b]).wait()

    def scatter_add_wait(cj, b):
        pltpu.async_copy(
            rows_v.at[b], acc_sh.at[dst_v.at[cj]], ssem.at[b], add=True
        ).wait()

    for h in range(NHALF):
        pltpu.sync_copy(src_hbm.at[c, s, pl.ds(h * HNCH, HNCH)], src_v)
        pltpu.sync_copy(dst_hbm.at[c, s, pl.ds(h * HNCH, HNCH)], dst_v)

        for b in range(NBUF):
            gather(b, b)

        @pl.loop(0, HNCH - NBUF, step=NBUF)
        def _(j):
            for b in range(NBUF):
                cj = j + b
                gather_wait(cj, b)
                scatter_add_wait(cj, b)
                gather(cj + NBUF, b)

        for b in range(NBUF):
            cj = HNCH - NBUF + b
            gather_wait(cj, b)
            scatter_add_wait(cj, b)

    plsc.subcore_barrier()

    @pl.when(s < NFULL)
    def _():
        pltpu.sync_copy(acc_sh.at[pl.ds(base, SLAB)], out_hbm.at[c, pl.ds(base, SLAB)])

    @pl.when(s == NFULL)
    def _():
        pltpu.sync_copy(
            acc_sh.at[pl.ds(NFULL * SLAB, REM)],
            out_hbm.at[c, pl.ds(NFULL * SLAB, REM)],
        )


@functools.partial(
    pl.kernel,
    out_type=jax.ShapeDtypeStruct((NC, N, D), jnp.float32),
    mesh=_mesh,
    scratch_types=[
        pltpu.VMEM((HNCH, K), jnp.int32),
        pltpu.VMEM((K, D), jnp.float32),
        pltpu.VMEM((ZROWS, D), jnp.float32),
        pltpu.VMEM_SHARED((N, D), jnp.float32),
        pltpu.SemaphoreType.DMA,
        pltpu.SemaphoreType.DMA((NBUF,)),
    ],
)
def _sc_degree(ones_hbm, dst_hbm, out_hbm,
               dst_v, rows_v, zbuf_v, acc_sh, gsem, ssem):
    c = lax.axis_index("c")
    s = lax.axis_index("s")
    base = pl.multiple_of(s * SLAB, 8)
    _zero_init(s, base, zbuf_v, acc_sh, gsem)
    pltpu.sync_copy(dst_hbm.at[c, s, pl.ds(0, HNCH)], dst_v)
    pltpu.async_copy(ones_hbm.at[dst_v.at[0]], rows_v, gsem).wait()
    plsc.subcore_barrier()

    def scatter_add(cj, b):
        pltpu.async_copy(rows_v, acc_sh.at[dst_v.at[cj]], ssem.at[b], add=True)

    def scatter_wait(cj, b):
        pltpu.make_async_copy(rows_v, acc_sh.at[dst_v.at[cj]], ssem.at[b]).wait()

    for h in range(NHALF):
        if h > 0:
            pltpu.sync_copy(dst_hbm.at[c, s, pl.ds(h * HNCH, HNCH)], dst_v)

        for b in range(NBUF):
            scatter_add(b, b)

        @pl.loop(0, HNCH - NBUF, step=NBUF)
        def _(j):
            for b in range(NBUF):
                cj = j + b
                scatter_wait(cj, b)
                scatter_add(cj + NBUF, b)

        for b in range(NBUF):
            scatter_wait(HNCH - NBUF + b, b)

    plsc.subcore_barrier()

    @pl.when(s < NFULL)
    def _():
        pltpu.sync_copy(acc_sh.at[pl.ds(base, SLAB)], out_hbm.at[c, pl.ds(base, SLAB)])

    @pl.when(s == NFULL)
    def _():
        pltpu.sync_copy(
            acc_sh.at[pl.ds(NFULL * SLAB, REM)],
            out_hbm.at[c, pl.ds(NFULL * SLAB, REM)],
        )


_R = 2000


def _pre_body(deg_ref, x_ref, w_ref, dinv_ref, g_ref):
    deg = deg_ref[0][:, :1] + deg_ref[1][:, :1]
    dinv = lax.rsqrt(deg + 1.0)
    h = jnp.dot(x_ref[...], w_ref[...], preferred_element_type=jnp.float32)
    dinv_ref[...] = dinv
    g_ref[...] = h * dinv


_pre_call = pl.pallas_call(
    _pre_body,
    grid=(N // _R,),
    in_specs=[
        pl.BlockSpec((NC, _R, D), lambda i: (0, i, 0)),
        pl.BlockSpec((_R, D), lambda i: (i, 0)),
        pl.BlockSpec((D, D), lambda i: (0, 0)),
    ],
    out_specs=[
        pl.BlockSpec((_R, 1), lambda i: (i, 0)),
        pl.BlockSpec((_R, D), lambda i: (i, 0)),
    ],
    out_shape=[
        jax.ShapeDtypeStruct((N, 1), jnp.float32),
        jax.ShapeDtypeStruct((N, D), jnp.float32),
    ],
)


def _mid_body(parts_ref, g_ref, dinv_ref, b_ref, w_ref, gn_ref):
    p = parts_ref[0] + parts_ref[1] + g_ref[...]
    y = jnp.maximum(dinv_ref[...] * p + b_ref[...], 0.0)
    gn = jnp.dot(y, w_ref[...], preferred_element_type=jnp.float32)
    gn_ref[...] = gn * dinv_ref[...]


_mid_call = pl.pallas_call(
    _mid_body,
    grid=(N // _R,),
    in_specs=[
        pl.BlockSpec((NC, _R, D), lambda i: (0, i, 0)),
        pl.BlockSpec((_R, D), lambda i: (i, 0)),
        pl.BlockSpec((_R, 1), lambda i: (i, 0)),
        pl.BlockSpec((1, D), lambda i: (0, 0)),
        pl.BlockSpec((D, D), lambda i: (0, 0)),
    ],
    out_specs=pl.BlockSpec((_R, D), lambda i: (i, 0)),
    out_shape=jax.ShapeDtypeStruct((N, D), jnp.float32),
)


def _post_body(parts_ref, g_ref, dinv_ref, b_ref, out_ref):
    p = parts_ref[0] + parts_ref[1] + g_ref[...]
    out_ref[...] = dinv_ref[...] * p + b_ref[...]


_post_call = pl.pallas_call(
    _post_body,
    grid=(N // _R,),
    in_specs=[
        pl.BlockSpec((NC, _R, D), lambda i: (0, i, 0)),
        pl.BlockSpec((_R, D), lambda i: (i, 0)),
        pl.BlockSpec((_R, 1), lambda i: (i, 0)),
        pl.BlockSpec((1, D), lambda i: (0, 0)),
    ],
    out_specs=pl.BlockSpec((_R, D), lambda i: (i, 0)),
    out_shape=jax.ShapeDtypeStruct((N, D), jnp.float32),
)


def kernel(x, edge_index, W1, b1, W2, b2, W3, b3):
    src = edge_index[0].reshape(NC, NS, NCH, K)
    dst = edge_index[1].reshape(NC, NS, NCH, K)
    ones_nd = jnp.ones((N, D), jnp.float32)

    degparts = _sc_degree(ones_nd, dst)
    dinv, g1 = _pre_call(degparts, x, W1)
    parts1 = _sc_aggregate(g1, src, dst)
    g2 = _mid_call(parts1, g1, dinv, b1.reshape(1, D), W2)
    parts2 = _sc_aggregate(g2, src, dst)
    g3 = _mid_call(parts2, g2, dinv, b2.reshape(1, D), W3)
    parts3 = _sc_aggregate(g3, src, dst)
    return _post_call(parts3, g3, dinv, b3.reshape(1, D))


# speedup vs baseline: 1.1224x; 1.0057x over previous
import functools

import jax
import jax.numpy as jnp
from jax import lax
from jax.experimental import pallas as pl
from jax.experimental.pallas import tpu as pltpu
from jax.experimental.pallas import tpu_sc as plsc

N = 10000
E = 320000
D = 128
NC = 2
NS = 16
K = 50
EPT = E // (NC * NS)
NCH = EPT // K
NHALF = 5
HNCH = NCH // NHALF
SLAB = 640
NFULL = N // SLAB
REM = N - NFULL * SLAB
NBUF = 4
ZROWS = 40

_mesh = plsc.VectorSubcoreMesh(core_axis_name="c", subcore_axis_name="s")


def _zero_init(s, base, zbuf_v, acc_sh, sem):
    zeros16 = jnp.zeros((16,), jnp.float32)

    def fill(i, carry):
        for kk in range(D // 16):
            zbuf_v[i, pl.ds(kk * 16, 16)] = zeros16
        return carry

    lax.fori_loop(0, ZROWS, fill, 0)
    nslab = SLAB // ZROWS
    nrem = REM // ZROWS

    @pl.when(s < NFULL)
    def _():
        for jb in range(0, nslab, 4):
            for j in range(jb, jb + 4):
                off = pl.multiple_of(base + j * ZROWS, 8)
                pltpu.async_copy(zbuf_v, acc_sh.at[pl.ds(off, ZROWS)], sem)
            for j in range(4):
                pltpu.make_async_copy(zbuf_v, acc_sh.at[pl.ds(base, ZROWS)], sem).wait()

    @pl.when(s == NFULL)
    def _():
        for jb in range(0, nrem, 5):
            for j in range(jb, jb + 5):
                off = NFULL * SLAB + j * ZROWS
                pltpu.async_copy(zbuf_v, acc_sh.at[pl.ds(off, ZROWS)], sem)
            for j in range(5):
                pltpu.make_async_copy(
                    zbuf_v, acc_sh.at[pl.ds(NFULL * SLAB, ZROWS)], sem
                ).wait()


@functools.partial(
    pl.kernel,
    out_type=jax.ShapeDtypeStruct((NC, N, D), jnp.float32),
    mesh=_mesh,
    scratch_types=[
        pltpu.VMEM((HNCH, K), jnp.int32),
        pltpu.VMEM((HNCH, K), jnp.int32),
        pltpu.VMEM((NBUF, K, D), jnp.float32),
        pltpu.VMEM((ZROWS, D), jnp.float32),
        pltpu.VMEM_SHARED((N, D), jnp.float32),
        pltpu.SemaphoreType.DMA((NBUF,)),
        pltpu.SemaphoreType.DMA((NBUF,)),
    ],
)
def _sc_aggregate(g_hbm, src_hbm, dst_hbm, out_hbm,
                  src_v, dst_v, rows_v, zbuf_v, acc_sh, gsem, ssem):
    c = lax.axis_index("c")
    s = lax.axis_index("s")
    base = pl.multiple_of(s * SLAB, 8)
    _zero_init(s, base, zbuf_v, acc_sh, gsem.at[0])
    plsc.subcore_barrier()

    def gather(cj, b):
        pltpu.async_copy(g_hbm.at[src_v.at[cj]], rows_v.at[b], gsem.at[b])

    def gather_wait(cj, b):
        pltpu.make_async_copy(g_hbm.at[src_v.at[cj]], rows_v.at[b], gsem.at[b]).wait()

    def scatter_add_wait(cj, b):
        pltpu.async_copy(
            rows_v.at[b], acc_sh.at[dst_v.at[cj]], ssem.at[b], add=True
        ).wait()

    for h in range(NHALF):
        pltpu.sync_copy(src_hbm.at[c, s, pl.ds(h * HNCH, HNCH)], src_v)
        pltpu.sync_copy(dst_hbm.at[c, s, pl.ds(h * HNCH, HNCH)], dst_v)

        for b in range(NBUF):
            gather(b, b)

        @pl.loop(0, HNCH - NBUF, step=NBUF)
        def _(j):
            for b in range(NBUF):
                cj = j + b
                gather_wait(cj, b)
                scatter_add_wait(cj, b)
                gather(cj + NBUF, b)

        for b in range(NBUF):
            cj = HNCH - NBUF + b
            gather_wait(cj, b)
            scatter_add_wait(cj, b)

    plsc.subcore_barrier()

    @pl.when(s < NFULL)
    def _():
        pltpu.sync_copy(acc_sh.at[pl.ds(base, SLAB)], out_hbm.at[c, pl.ds(base, SLAB)])

    @pl.when(s == NFULL)
    def _():
        pltpu.sync_copy(
            acc_sh.at[pl.ds(NFULL * SLAB, REM)],
            out_hbm.at[c, pl.ds(NFULL * SLAB, REM)],
        )


@functools.partial(
    pl.kernel,
    out_type=jax.ShapeDtypeStruct((NC, N, D), jnp.float32),
    mesh=_mesh,
    scratch_types=[
        pltpu.VMEM((HNCH, K), jnp.int32),
        pltpu.VMEM((K, D), jnp.float32),
        pltpu.VMEM((ZROWS, D), jnp.float32),
        pltpu.VMEM_SHARED((N, D), jnp.float32),
        pltpu.SemaphoreType.DMA,
        pltpu.SemaphoreType.DMA((NBUF,)),
    ],
)
def _sc_degree(dst_hbm, out_hbm,
               dst_v, rows_v, zbuf_v, acc_sh, gsem, ssem):
    c = lax.axis_index("c")
    s = lax.axis_index("s")
    base = pl.multiple_of(s * SLAB, 8)
    _zero_init(s, base, zbuf_v, acc_sh, gsem)
    pltpu.sync_copy(dst_hbm.at[c, s, pl.ds(0, HNCH)], dst_v)
    ones16 = jnp.ones((16,), jnp.float32)

    def fill_ones(i, carry):
        for kk in range(D // 16):
            rows_v[i, pl.ds(kk * 16, 16)] = ones16
        return carry

    lax.fori_loop(0, K, fill_ones, 0)
    plsc.subcore_barrier()

    def scatter_add(cj, b):
        pltpu.async_copy(rows_v, acc_sh.at[dst_v.at[cj]], ssem.at[b], add=True)

    def scatter_wait(cj, b):
        pltpu.make_async_copy(rows_v, acc_sh.at[dst_v.at[cj]], ssem.at[b]).wait()

    for h in range(NHALF):
        if h > 0:
            pltpu.sync_copy(dst_hbm.at[c, s, pl.ds(h * HNCH, HNCH)], dst_v)

        for b in range(NBUF):
            scatter_add(b, b)

        @pl.loop(0, HNCH - NBUF, step=NBUF)
        def _(j):
            for b in range(NBUF):
                cj = j + b
                scatter_wait(cj, b)
                scatter_add(cj + NBUF, b)

        for b in range(NBUF):
            scatter_wait(HNCH - NBUF + b, b)

    plsc.subcore_barrier()

    @pl.when(s < NFULL)
    def _():
        pltpu.sync_copy(acc_sh.at[pl.ds(base, SLAB)], out_hbm.at[c, pl.ds(base, SLAB)])

    @pl.when(s == NFULL)
    def _():
        pltpu.sync_copy(
            acc_sh.at[pl.ds(NFULL * SLAB, REM)],
            out_hbm.at[c, pl.ds(NFULL * SLAB, REM)],
        )


_R = 2000


def _pre_body(deg_ref, x_ref, w_ref, dinv_ref, g_ref):
    deg = deg_ref[0][:, :1] + deg_ref[1][:, :1]
    dinv = lax.rsqrt(deg + 1.0)
    h = jnp.dot(x_ref[...], w_ref[...], preferred_element_type=jnp.float32)
    dinv_ref[...] = dinv
    g_ref[...] = h * dinv


_pre_call = pl.pallas_call(
    _pre_body,
    grid=(N // _R,),
    in_specs=[
        pl.BlockSpec((NC, _R, D), lambda i: (0, i, 0)),
        pl.BlockSpec((_R, D), lambda i: (i, 0)),
        pl.BlockSpec((D, D), lambda i: (0, 0)),
    ],
    out_specs=[
        pl.BlockSpec((_R, 1), lambda i: (i, 0)),
        pl.BlockSpec((_R, D), lambda i: (i, 0)),
    ],
    out_shape=[
        jax.ShapeDtypeStruct((N, 1), jnp.float32),
        jax.ShapeDtypeStruct((N, D), jnp.float32),
    ],
)


def _mid_body(parts_ref, g_ref, dinv_ref, b_ref, w_ref, gn_ref):
    p = parts_ref[0] + parts_ref[1] + g_ref[...]
    y = jnp.maximum(dinv_ref[...] * p + b_ref[...], 0.0)
    gn = jnp.dot(y, w_ref[...], preferred_element_type=jnp.float32)
    gn_ref[...] = gn * dinv_ref[...]


_mid_call = pl.pallas_call(
    _mid_body,
    grid=(N // _R,),
    in_specs=[
        pl.BlockSpec((NC, _R, D), lambda i: (0, i, 0)),
        pl.BlockSpec((_R, D), lambda i: (i, 0)),
        pl.BlockSpec((_R, 1), lambda i: (i, 0)),
        pl.BlockSpec((1, D), lambda i: (0, 0)),
        pl.BlockSpec((D, D), lambda i: (0, 0)),
    ],
    out_specs=pl.BlockSpec((_R, D), lambda i: (i, 0)),
    out_shape=jax.ShapeDtypeStruct((N, D), jnp.float32),
)


def _post_body(parts_ref, g_ref, dinv_ref, b_ref, out_ref):
    p = parts_ref[0] + parts_ref[1] + g_ref[...]
    out_ref[...] = dinv_ref[...] * p + b_ref[...]


_post_call = pl.pallas_call(
    _post_body,
    grid=(N // _R,),
    in_specs=[
        pl.BlockSpec((NC, _R, D), lambda i: (0, i, 0)),
        pl.BlockSpec((_R, D), lambda i: (i, 0)),
        pl.BlockSpec((_R, 1), lambda i: (i, 0)),
        pl.BlockSpec((1, D), lambda i: (0, 0)),
    ],
    out_specs=pl.BlockSpec((_R, D), lambda i: (i, 0)),
    out_shape=jax.ShapeDtypeStruct((N, D), jnp.float32),
)


def kernel(x, edge_index, W1, b1, W2, b2, W3, b3):
    src = edge_index[0].reshape(NC, NS, NCH, K)
    dst = edge_index[1].reshape(NC, NS, NCH, K)
    degparts = _sc_degree(dst)
    dinv, g1 = _pre_call(degparts, x, W1)
    parts1 = _sc_aggregate(g1, src, dst)
    g2 = _mid_call(parts1, g1, dinv, b1.reshape(1, D), W2)
    parts2 = _sc_aggregate(g2, src, dst)
    g3 = _mid_call(parts2, g2, dinv, b2.reshape(1, D), W3)
    parts3 = _sc_aggregate(g3, src, dst)
    return _post_call(parts3, g3, dinv, b3.reshape(1, D))
